# baseline (speedup 1.0000x reference)
import functools

import jax
import jax.numpy as jnp
from jax import lax
from jax.experimental import pallas as pl
from jax.experimental.pallas import tpu as pltpu
from jax.experimental.pallas import tpu_sc as plsc

N_NODES = 10000
N_SAMPLE = 32
D_FEAT = 128
LANES = 16

NW = 32
R = 312
TAIL = N_NODES - NW * R
TAIL_OFF = NW * N_SAMPLE * R
IDX_T0 = N_SAMPLE * R
HBM_CHUNKS = ((0, 128), (256, 56))
SPM_CHUNKS = ((128, 128),)
ROWS_PER_TILE = 624
STAGE_TAIL = N_NODES - 16 * ROWS_PER_TILE


def _make_kernel():
    mesh = plsc.VectorSubcoreMesh(core_axis_name="c", subcore_axis_name="s")

    @functools.partial(
        pl.kernel,
        mesh=mesh,
        out_type=jax.ShapeDtypeStruct((N_NODES, D_FEAT), jnp.float32),
        scratch_types=[
            pltpu.VMEM((N_SAMPLE * R,), jnp.int32),
            pltpu.VMEM((N_SAMPLE * TAIL,), jnp.int32),
            pltpu.VMEM((R, D_FEAT), jnp.float32),
            pltpu.VMEM_SHARED((N_NODES, D_FEAT), jnp.float32),
            pltpu.SemaphoreType.DMA,
            pltpu.SemaphoreType.DMA,
            pltpu.SemaphoreType.DMA,
            pltpu.SemaphoreType.DMA,
            pltpu.SemaphoreType.DMA,
            pltpu.SemaphoreType.DMA,
            pltpu.SemaphoreType.DMA,
        ],
    )
    def run(feat_hbm, nlw_hbm, nlt_hbm, out_hbm, idx_v, idxt_v, acc_v,
            table_s, sem_t, sem_i, sem_c0, sem_c1, sem_c2, sem_tl, sem_o):
        num_cores = 2
        sid = lax.axis_index("s")
        wid = sid * num_cores + lax.axis_index("c")
        base = wid * R
        csem = (sem_c0, sem_c1, sem_c2)
        is_tail = wid == NW - 1

        pltpu.async_copy(
            feat_hbm.at[pl.ds(sid * ROWS_PER_TILE, ROWS_PER_TILE)],
            table_s.at[pl.ds(sid * ROWS_PER_TILE, ROWS_PER_TILE)],
            sem_t,
        )

        @pl.when(sid == 15)
        def _():
            pltpu.async_copy(
                feat_hbm.at[pl.ds(16 * ROWS_PER_TILE, STAGE_TAIL)],
                table_s.at[pl.ds(16 * ROWS_PER_TILE, STAGE_TAIL)],
                sem_t,
            )

        pltpu.async_copy(
            nlw_hbm.at[pl.ds(wid * (N_SAMPLE * R), N_SAMPLE * R)], idx_v, sem_i
        )

        @pl.when(is_tail)
        def _():
            pltpu.async_copy(nlt_hbm, idxt_v, sem_i)

        zeros = jnp.zeros((LANES,), jnp.float32)

        def zero_body(r, carry):
            for k in range(D_FEAT // LANES):
                acc_v[r, pl.ds(k * LANES, LANES)] = zeros
            return carry

        lax.fori_loop(0, R, zero_body, 0)

        pltpu.make_async_copy(
            nlw_hbm.at[pl.ds(wid * (N_SAMPLE * R), N_SAMPLE * R)], idx_v, sem_i
        ).wait()

        @pl.when(is_tail)
        def _():
            pltpu.make_async_copy(nlt_hbm, idxt_v, sem_i).wait()

        for ci, (c0, ln) in enumerate(HBM_CHUNKS):
            for j in range(N_SAMPLE):
                pltpu.async_copy(
                    feat_hbm.at[idx_v.at[pl.ds(j * R + c0, ln)]],
                    acc_v.at[pl.ds(c0, ln)],
                    csem[ci],
                    add=True,
                )

        pltpu.make_async_copy(
            feat_hbm.at[pl.ds(sid * ROWS_PER_TILE, ROWS_PER_TILE)],
            table_s.at[pl.ds(sid * ROWS_PER_TILE, ROWS_PER_TILE)],
            sem_t,
        ).wait()

        @pl.when(sid == 15)
        def _():
            pltpu.make_async_copy(
                feat_hbm.at[pl.ds(16 * ROWS_PER_TILE, STAGE_TAIL)],
                table_s.at[pl.ds(16 * ROWS_PER_TILE, STAGE_TAIL)],
                sem_t,
            ).wait()

        plsc.subcore_barrier()

        for ci, (c0, ln) in enumerate(SPM_CHUNKS):
            for j in range(N_SAMPLE):
                pltpu.async_copy(
                    table_s.at[idx_v.at[pl.ds(j * R + c0, ln)]],
                    acc_v.at[pl.ds(c0, ln)],
                    csem[2 + ci],
                    add=True,
                )

        all_chunks = tuple(
            (c0, ln, feat_hbm, csem[ci])
            for ci, (c0, ln) in enumerate(HBM_CHUNKS)
        ) + tuple(
            (c0, ln, table_s, csem[2 + ci])
            for ci, (c0, ln) in enumerate(SPM_CHUNKS)
        )
        for c0, ln, src_ref, sem in all_chunks:
            for j in range(N_SAMPLE):
                pltpu.make_async_copy(
                    src_ref.at[idx_v.at[pl.ds(j * R + c0, ln)]],
                    acc_v.at[pl.ds(c0, ln)],
                    sem,
                ).wait()

            def scale_body(r, carry, c0=c0):
                for k in range(D_FEAT // LANES):
                    acc_v[c0 + r, pl.ds(k * LANES, LANES)] = acc_v[
                        c0 + r, pl.ds(k * LANES, LANES)
                    ] * (1.0 / N_SAMPLE)
                return carry

            lax.fori_loop(0, ln, scale_body, 0)
            pltpu.async_copy(
                acc_v.at[pl.ds(c0, ln)], out_hbm.at[pl.ds(base + c0, ln)], sem_o
            )

        for c0, ln, src_ref, sem in all_chunks:
            pltpu.make_async_copy(
                acc_v.at[pl.ds(c0, ln)], out_hbm.at[pl.ds(base + c0, ln)], sem_o
            ).wait()

        @pl.when(is_tail)
        def _():
            def tz_body(r, carry):
                for k in range(D_FEAT // LANES):
                    acc_v[r, pl.ds(k * LANES, LANES)] = zeros
                return carry

            lax.fori_loop(0, TAIL, tz_body, 0)
            for j in range(N_SAMPLE):
                pltpu.async_copy(
                    feat_hbm.at[idxt_v.at[pl.ds(j * TAIL, TAIL)]],
                    acc_v.at[pl.ds(0, TAIL)],
                    sem_tl,
                    add=True,
                )
            for j in range(N_SAMPLE):
                pltpu.make_async_copy(
                    feat_hbm.at[idxt_v.at[pl.ds(j * TAIL, TAIL)]],
                    acc_v.at[pl.ds(0, TAIL)],
                    sem_tl,
                ).wait()

            def tail_scale(r, carry):
                for k in range(D_FEAT // LANES):
                    acc_v[r, pl.ds(k * LANES, LANES)] = acc_v[
                        r, pl.ds(k * LANES, LANES)
                    ] * (1.0 / N_SAMPLE)
                return carry

            lax.fori_loop(0, TAIL, tail_scale, 0)
            pltpu.sync_copy(
                acc_v.at[pl.ds(0, TAIL)], out_hbm.at[pl.ds(NW * R, TAIL)]
            )

    return run


_kernel = _make_kernel()


def kernel(feature, neighbor_list):
    main = neighbor_list[: NW * R].reshape(NW, R, N_SAMPLE)
    main = main.transpose(0, 2, 1).reshape(-1)
    tail = neighbor_list[NW * R :].T.reshape(-1)
    return _kernel(feature, main, tail)


# speedup vs baseline: 1.0275x; 1.0275x over previous
import functools

import jax
import jax.numpy as jnp
from jax import lax
from jax.experimental import pallas as pl
from jax.experimental.pallas import tpu as pltpu
from jax.experimental.pallas import tpu_sc as plsc

N_NODES = 10000
N_SAMPLE = 32
D_FEAT = 128
LANES = 16

NW = 32
R = 312
TAIL = N_NODES - NW * R
TAIL_OFF = NW * N_SAMPLE * R
IDX_T0 = N_SAMPLE * R
HBM_CHUNKS = ((0, 128),)
SPM_CHUNKS = ((128, 128), (256, 56))
ROWS_PER_TILE = 624
STAGE_TAIL = N_NODES - 16 * ROWS_PER_TILE


def _make_kernel():
    mesh = plsc.VectorSubcoreMesh(core_axis_name="c", subcore_axis_name="s")

    @functools.partial(
        pl.kernel,
        mesh=mesh,
        out_type=jax.ShapeDtypeStruct((N_NODES, D_FEAT), jnp.float32),
        scratch_types=[
            pltpu.VMEM((N_SAMPLE * R,), jnp.int32),
            pltpu.VMEM((N_SAMPLE * TAIL,), jnp.int32),
            pltpu.VMEM((R, D_FEAT), jnp.float32),
            pltpu.VMEM_SHARED((N_NODES, D_FEAT), jnp.float32),
            pltpu.SemaphoreType.DMA,
            pltpu.SemaphoreType.DMA,
            pltpu.SemaphoreType.DMA,
            pltpu.SemaphoreType.DMA,
            pltpu.SemaphoreType.DMA,
            pltpu.SemaphoreType.DMA,
            pltpu.SemaphoreType.DMA,
        ],
    )
    def run(feat_hbm, nlw_hbm, nlt_hbm, out_hbm, idx_v, idxt_v, acc_v,
            table_s, sem_t, sem_i, sem_c0, sem_c1, sem_c2, sem_tl, sem_o):
        num_cores = 2
        sid = lax.axis_index("s")
        wid = sid * num_cores + lax.axis_index("c")
        base = wid * R
        csem = (sem_c0, sem_c1, sem_c2)
        is_tail = wid == NW - 1

        pltpu.async_copy(
            feat_hbm.at[pl.ds(sid * ROWS_PER_TILE, ROWS_PER_TILE)],
            table_s.at[pl.ds(sid * ROWS_PER_TILE, ROWS_PER_TILE)],
            sem_t,
        )

        @pl.when(sid == 15)
        def _():
            pltpu.async_copy(
                feat_hbm.at[pl.ds(16 * ROWS_PER_TILE, STAGE_TAIL)],
                table_s.at[pl.ds(16 * ROWS_PER_TILE, STAGE_TAIL)],
                sem_t,
            )

        pltpu.async_copy(
            nlw_hbm.at[pl.ds(wid * (N_SAMPLE * R), N_SAMPLE * R)], idx_v, sem_i
        )

        @pl.when(is_tail)
        def _():
            pltpu.async_copy(nlt_hbm, idxt_v, sem_i)

        zeros = jnp.zeros((LANES,), jnp.float32)

        def zero_body(r, carry):
            for k in range(D_FEAT // LANES):
                acc_v[r, pl.ds(k * LANES, LANES)] = zeros
            return carry

        lax.fori_loop(0, R, zero_body, 0)

        pltpu.make_async_copy(
            nlw_hbm.at[pl.ds(wid * (N_SAMPLE * R), N_SAMPLE * R)], idx_v, sem_i
        ).wait()

        @pl.when(is_tail)
        def _():
            pltpu.make_async_copy(nlt_hbm, idxt_v, sem_i).wait()

        for ci, (c0, ln) in enumerate(HBM_CHUNKS):
            for j in range(N_SAMPLE):
                pltpu.async_copy(
                    feat_hbm.at[idx_v.at[pl.ds(j * R + c0, ln)]],
                    acc_v.at[pl.ds(c0, ln)],
                    csem[ci],
                    add=True,
                )

        pltpu.make_async_copy(
            feat_hbm.at[pl.ds(sid * ROWS_PER_TILE, ROWS_PER_TILE)],
            table_s.at[pl.ds(sid * ROWS_PER_TILE, ROWS_PER_TILE)],
            sem_t,
        ).wait()

        @pl.when(sid == 15)
        def _():
            pltpu.make_async_copy(
                feat_hbm.at[pl.ds(16 * ROWS_PER_TILE, STAGE_TAIL)],
                table_s.at[pl.ds(16 * ROWS_PER_TILE, STAGE_TAIL)],
                sem_t,
            ).wait()

        plsc.subcore_barrier()

        for ci, (c0, ln) in enumerate(SPM_CHUNKS):
            for j in range(N_SAMPLE):
                pltpu.async_copy(
                    table_s.at[idx_v.at[pl.ds(j * R + c0, ln)]],
                    acc_v.at[pl.ds(c0, ln)],
                    csem[1 + ci],
                    add=True,
                )

        all_chunks = tuple(
            (c0, ln, feat_hbm, csem[ci])
            for ci, (c0, ln) in enumerate(HBM_CHUNKS)
        ) + tuple(
            (c0, ln, table_s, csem[1 + ci])
            for ci, (c0, ln) in enumerate(SPM_CHUNKS)
        )
        for c0, ln, src_ref, sem in all_chunks:
            for j in range(N_SAMPLE):
                pltpu.make_async_copy(
                    src_ref.at[idx_v.at[pl.ds(j * R + c0, ln)]],
                    acc_v.at[pl.ds(c0, ln)],
                    sem,
                ).wait()

            def scale_body(r, carry, c0=c0):
                for k in range(D_FEAT // LANES):
                    acc_v[c0 + r, pl.ds(k * LANES, LANES)] = acc_v[
                        c0 + r, pl.ds(k * LANES, LANES)
                    ] * (1.0 / N_SAMPLE)
                return carry

            lax.fori_loop(0, ln, scale_body, 0)
            pltpu.async_copy(
                acc_v.at[pl.ds(c0, ln)], out_hbm.at[pl.ds(base + c0, ln)], sem_o
            )

        for c0, ln, src_ref, sem in all_chunks:
            pltpu.make_async_copy(
                acc_v.at[pl.ds(c0, ln)], out_hbm.at[pl.ds(base + c0, ln)], sem_o
            ).wait()

        @pl.when(is_tail)
        def _():
            def tz_body(r, carry):
                for k in range(D_FEAT // LANES):
                    acc_v[r, pl.ds(k * LANES, LANES)] = zeros
                return carry

            lax.fori_loop(0, TAIL, tz_body, 0)
            for j in range(N_SAMPLE):
                pltpu.async_copy(
                    feat_hbm.at[idxt_v.at[pl.ds(j * TAIL, TAIL)]],
                    acc_v.at[pl.ds(0, TAIL)],
                    sem_tl,
                    add=True,
                )
            for j in range(N_SAMPLE):
                pltpu.make_async_copy(
                    feat_hbm.at[idxt_v.at[pl.ds(j * TAIL, TAIL)]],
                    acc_v.at[pl.ds(0, TAIL)],
                    sem_tl,
                ).wait()

            def tail_scale(r, carry):
                for k in range(D_FEAT // LANES):
                    acc_v[r, pl.ds(k * LANES, LANES)] = acc_v[
                        r, pl.ds(k * LANES, LANES)
                    ] * (1.0 / N_SAMPLE)
                return carry

            lax.fori_loop(0, TAIL, tail_scale, 0)
            pltpu.sync_copy(
                acc_v.at[pl.ds(0, TAIL)], out_hbm.at[pl.ds(NW * R, TAIL)]
            )

    return run


_kernel = _make_kernel()


def kernel(feature, neighbor_list):
    main = neighbor_list[: NW * R].reshape(NW, R, N_SAMPLE)
    main = main.transpose(0, 2, 1).reshape(-1)
    tail = neighbor_list[NW * R :].T.reshape(-1)
    return _kernel(feature, main, tail)
